# Initial kernel scaffold; baseline (speedup 1.0000x reference)
import jax
import jax.numpy as jnp
from jax.experimental import pallas as pl


def kernel(image, centroids, cluster_labels):
    raise NotImplementedError("write your pallas kernel here")


# speedup vs baseline: 192.8952x; 192.8952x over previous
import functools

import jax
import jax.numpy as jnp
from jax import lax
from jax.experimental import pallas as pl
from jax.experimental.pallas import tpu as pltpu
from jax.experimental.pallas import tpu_sc as plsc

BS = 32
P = 196
D = 32
K = 512
PS = 16
NROW = 14

NC = 2
NS = 16
NW = NC * NS
IMG = NROW * PS


def _assign_body(img_ref, cent_ref, out_ref):
    x = img_ref[0]
    c = cent_ref[...]
    dot = jnp.dot(x, c, preferred_element_type=jnp.float32,
                  precision=lax.Precision.HIGHEST)
    cn = jnp.sum(c * c, axis=0, keepdims=True)
    score = cn - 2.0 * dot
    m = jnp.max(score, axis=1, keepdims=True)
    ids = lax.broadcasted_iota(jnp.int32, (P, K), 1)
    a = jnp.min(jnp.where(score >= m, ids, K), axis=1)
    out_ref[...] = a.reshape(1, 1, P)


def _assignment(image, centroids):
    return pl.pallas_call(
        _assign_body,
        grid=(BS,),
        in_specs=[
            pl.BlockSpec((1, P, D), lambda b: (b, 0, 0)),
            pl.BlockSpec((D, K), lambda b: (0, 0)),
        ],
        out_specs=pl.BlockSpec((1, 1, P), lambda b: (b, 0, 0)),
        out_shape=jax.ShapeDtypeStruct((BS, 1, P), jnp.int32),
    )(image, centroids)


@functools.cache
def _sc_gather_kernel():
    mesh = plsc.VectorSubcoreMesh(core_axis_name="c", subcore_axis_name="s")

    @functools.partial(
        pl.kernel,
        mesh=mesh,
        out_type=jax.ShapeDtypeStruct((BS, IMG, IMG), jnp.int32),
        scratch_types=[
            pltpu.VMEM((NROW, PS), jnp.int32),
            pltpu.VMEM((PS, 256), jnp.int32),
            pltpu.VMEM((PS, IMG), jnp.int32),
            pltpu.SemaphoreType.DMA,
        ],
    )
    def _sc_gather(table_hbm, assign_hbm, out_hbm, a_v, patches_v, rows_v, sem):
        wid = lax.axis_index("s") * NC + lax.axis_index("c")
        pltpu.sync_copy(assign_hbm.at[wid], a_v)

        def body(r, carry):
            pltpu.async_copy(table_hbm.at[a_v.at[r]], patches_v, sem).wait()
            for i in range(PS):
                for c in range(NROW):
                    rows_v[i, pl.ds(c * PS, PS)] = patches_v[c, pl.ds(i * PS, PS)]
            pltpu.sync_copy(rows_v, out_hbm.at[wid, pl.ds(r * PS, PS)])
            return carry

        lax.fori_loop(0, NROW, body, 0)

    return _sc_gather


def kernel(image, centroids, cluster_labels):
    assign = _assignment(image, centroids).reshape(BS, NROW, NROW)
    assign = jnp.pad(assign, ((0, 0), (0, 0), (0, PS - NROW)))
    table = jnp.transpose(cluster_labels)
    return _sc_gather_kernel()(table, assign)


# speedup vs baseline: 193.2704x; 1.0019x over previous
import functools

import jax
import jax.numpy as jnp
from jax import lax
from jax.experimental import pallas as pl
from jax.experimental.pallas import tpu as pltpu
from jax.experimental.pallas import tpu_sc as plsc

BS = 32
P = 196
D = 32
K = 512
PS = 16
NROW = 14

NC = 2
NS = 16
NW = NC * NS
IMG = NROW * PS


def _assign_body(img_ref, cent_ref, out_ref):
    x = img_ref[0]
    c = cent_ref[...]
    dot = jnp.dot(x, c, preferred_element_type=jnp.float32,
                  precision=lax.Precision.HIGHEST)
    cn = jnp.sum(c * c, axis=0, keepdims=True)
    score = cn - 2.0 * dot
    m = jnp.max(score, axis=1, keepdims=True)
    ids = lax.broadcasted_iota(jnp.int32, (P, K), 1)
    a = jnp.min(jnp.where(score >= m, ids, K), axis=1)
    out_ref[...] = a.reshape(1, 1, P)


def _assignment(image, centroids):
    return pl.pallas_call(
        _assign_body,
        grid=(BS,),
        in_specs=[
            pl.BlockSpec((1, P, D), lambda b: (b, 0, 0)),
            pl.BlockSpec((D, K), lambda b: (0, 0)),
        ],
        out_specs=pl.BlockSpec((1, 1, P), lambda b: (b, 0, 0)),
        out_shape=jax.ShapeDtypeStruct((BS, 1, P), jnp.int32),
    )(image, centroids)


@functools.cache
def _sc_gather_kernel():
    mesh = plsc.VectorSubcoreMesh(core_axis_name="c", subcore_axis_name="s")

    @functools.partial(
        pl.kernel,
        mesh=mesh,
        out_type=jax.ShapeDtypeStruct((BS, IMG, IMG), jnp.int32),
        scratch_types=[
            pltpu.VMEM((2, 112), jnp.int32),
            pltpu.VMEM((IMG, 256), jnp.int32),
            pltpu.VMEM((IMG, IMG), jnp.int32),
            pltpu.SemaphoreType.DMA,
        ],
    )
    def _sc_gather(table_hbm, assign_hbm, out_hbm, a_v, patches_v, out_v, sem):
        wid = lax.axis_index("s") * NC + lax.axis_index("c")
        pltpu.sync_copy(assign_hbm.at[wid], a_v)
        cp0 = pltpu.async_copy(table_hbm.at[a_v.at[0]],
                               patches_v.at[pl.ds(0, 112)], sem)
        cp1 = pltpu.async_copy(table_hbm.at[a_v.at[1]],
                               patches_v.at[pl.ds(112, 112)], sem)
        cp0.wait()
        cp1.wait()

        def body(r, carry):
            r16 = r * PS
            for i in range(PS):
                for c in range(NROW):
                    out_v[r16 + i, pl.ds(c * PS, PS)] = (
                        patches_v[r16 + c, pl.ds(i * PS, PS)])
            return carry

        lax.fori_loop(0, NROW, body, 0)
        pltpu.sync_copy(out_v, out_hbm.at[wid])

    return _sc_gather


def kernel(image, centroids, cluster_labels):
    assign = _assignment(image, centroids).reshape(BS, NROW, NROW)
    assign = jnp.pad(assign, ((0, 0), (0, 0), (0, PS - NROW)))
    assign = assign.reshape(BS, 2, 112)
    table = jnp.transpose(cluster_labels)
    return _sc_gather_kernel()(table, assign)
